# baseline (speedup 1.0000x reference)
import functools

import jax
import jax.numpy as jnp
from jax import lax
from jax.experimental import pallas as pl
from jax.experimental.pallas import tpu as pltpu
from jax.experimental.pallas import tpu_sc as plsc

T = 2048
E = 8
HID = 2048
INTER = 1408
K = 2
BT = 256
P = T * K
S = P + E * BT
NBLK = S // BT
NBPAD = 32
NW = 32
PW = P // NW
CH = 16
NCH = PW // CH
HW = HID // 2

_f32 = jnp.float32


def _routing_body(logits_ref, x_ref, dest_ref, wk_ref, be_ref, xp_ref):
    x = x_ref[...]
    xp_ref[...] = pltpu.pack_elementwise([x[:, :HW], x[:, HW:]],
                                         packed_dtype=jnp.bfloat16)
    l = logits_ref[...]
    io = lax.broadcasted_iota(jnp.int32, (T, E), 1)
    m1 = jnp.max(l, axis=1, keepdims=True)
    sel1 = io == jnp.min(jnp.where(l == m1, io, E), axis=1, keepdims=True)
    lm = jnp.where(sel1, -jnp.inf, l)
    m2 = jnp.max(lm, axis=1, keepdims=True)
    sel2 = io == jnp.min(jnp.where(lm == m2, io, E), axis=1, keepdims=True)
    w0 = jax.nn.sigmoid(m1 - m2)
    wk_ref[...] = jnp.concatenate([w0, 1.0 - w0], axis=1)

    oh = jnp.concatenate([sel1.astype(_f32), sel2.astype(_f32)], axis=0)
    c = 128
    ri = lax.broadcasted_iota(jnp.int32, (c, c), 0)
    ci = lax.broadcasted_iota(jnp.int32, (c, c), 1)
    ltri = (ri > ci).astype(_f32)
    ones_row = jnp.ones((1, c), _f32)
    running = jnp.zeros((1, E), _f32)
    chunks = []
    for i in range(P // c):
        oc = lax.slice(oh, (i * c, 0), ((i + 1) * c, E))
        chunks.append(jnp.dot(ltri, oc, preferred_element_type=_f32) + running)
        running = running + jnp.dot(ones_row, oc, preferred_element_type=_f32)
    prefix = jnp.concatenate(chunks, axis=0)
    cap = jnp.ceil(running / BT) * BT
    a_lt_b = (lax.broadcasted_iota(jnp.int32, (E, E), 0)
              < lax.broadcasted_iota(jnp.int32, (E, E), 1)).astype(_f32)
    start = jnp.dot(cap, a_lt_b, preferred_element_type=_f32)
    dest = jnp.sum(oh * (start + prefix), axis=1, keepdims=True)
    dest_ref[...] = dest.astype(jnp.int32)
    posb = (lax.broadcasted_iota(jnp.int32, (NBPAD, E), 0) * BT).astype(_f32)
    eiof = lax.broadcasted_iota(jnp.int32, (NBPAD, E), 1).astype(_f32)
    bef = jnp.sum((start <= posb).astype(_f32), axis=1, keepdims=True) - 1.0
    present = (running > 0).astype(_f32)
    rank = jnp.dot(present, a_lt_b, preferred_element_type=_f32)
    runidx = jnp.sum(present * (eiof < bef).astype(_f32), axis=1, keepdims=True)
    parity = runidx - 2.0 * jnp.floor(runidx / 2.0)
    startb = jnp.sum((eiof == bef).astype(_f32) * start, axis=1, keepdims=True)
    posb0 = posb[:, 0:1]
    tot = jnp.sum(cap, axis=1, keepdims=True)
    used = (posb0 < tot).astype(_f32)
    first = ((startb == posb0).astype(_f32)) * used
    sel_nxt = ((rank == runidx + 1.0).astype(_f32)) * present
    has = jnp.sum(sel_nxt, axis=1, keepdims=True)
    nxt = jnp.sum(sel_nxt * eiof, axis=1, keepdims=True) - (1.0 - has)
    zero = jnp.zeros((NBPAD, 3), _f32)
    meta = jnp.concatenate([bef, first, parity, nxt, used, zero], axis=1)
    be_ref[...] = meta.astype(jnp.int32)


_routing = pl.pallas_call(
    _routing_body,
    out_shape=(jax.ShapeDtypeStruct((P, 1), jnp.int32),
               jax.ShapeDtypeStruct((T, K), _f32),
               jax.ShapeDtypeStruct((NBPAD, 8), jnp.int32),
               jax.ShapeDtypeStruct((T, HW), jnp.int32)),
)


def _gemm1_body(m_ref, xg_ref, w13_hbm, a_ref, wbuf, sems):
    i = pl.program_id(0)
    cur = m_ref[i, 0]
    first = m_ref[i, 1]
    pe = m_ref[i, 2]
    nxt = m_ref[i, 3]

    def w13_copies(e, slot):
        return [pltpu.make_async_copy(w13_hbm.at[e, pl.ds(lo, INTER)],
                                      wbuf.at[slot, pl.ds(lo, INTER)],
                                      sems.at[slot, q])
                for q, lo in ((0, 0), (1, INTER))]

    @pl.when(i == 0)
    def _():
        for c in w13_copies(cur, 0):
            c.start()

    @pl.when(first == 1)
    def _():
        @pl.when(nxt >= 0)
        def _():
            for c in w13_copies(nxt, 1 - pe):
                c.start()
        for c in w13_copies(cur, pe):
            c.wait()

    xi = xg_ref[...]
    x = jnp.concatenate(
        [pltpu.unpack_elementwise(xi, index=0, packed_dtype=jnp.bfloat16,
                                  unpacked_dtype=_f32),
         pltpu.unpack_elementwise(xi, index=1, packed_dtype=jnp.bfloat16,
                                  unpacked_dtype=_f32)], axis=1)
    h = lax.dot_general(x, wbuf[pe], (((1,), (1,)), ((), ())),
                        preferred_element_type=_f32,
                        precision=lax.Precision.DEFAULT)
    g = h[:, :INTER]
    u = h[:, INTER:]
    a_ref[...] = (g * jax.nn.sigmoid(g) * u).astype(jnp.bfloat16)


_gemm1 = pl.pallas_call(
    _gemm1_body,
    grid_spec=pltpu.PrefetchScalarGridSpec(
        num_scalar_prefetch=1,
        grid=(NBLK,),
        in_specs=[
            pl.BlockSpec((BT, HW), lambda i, m: (i, 0)),
            pl.BlockSpec(memory_space=pl.ANY),
        ],
        out_specs=pl.BlockSpec((BT, INTER), lambda i, m: (i, 0)),
        scratch_shapes=[pltpu.VMEM((2, 2 * INTER, HID), _f32),
                        pltpu.SemaphoreType.DMA((2, 2))],
    ),
    out_shape=jax.ShapeDtypeStruct((S, INTER), jnp.bfloat16),
)


def _gemm2_body(m_ref, a_ref, w2_hbm, out_ref, wbuf, sems):
    i = pl.program_id(0)
    cur = m_ref[i, 0]
    first = m_ref[i, 1]
    pe = m_ref[i, 2]
    nxt = m_ref[i, 3]

    def w2_copies(e, slot):
        return [pltpu.make_async_copy(w2_hbm.at[e, pl.ds(lo, HID // 2)],
                                      wbuf.at[slot, pl.ds(lo, HID // 2)],
                                      sems.at[slot, q])
                for q, lo in ((0, 0), (1, HID // 2))]

    @pl.when(i == 0)
    def _():
        for c in w2_copies(cur, 0):
            c.start()

    @pl.when(first == 1)
    def _():
        @pl.when(nxt >= 0)
        def _():
            for c in w2_copies(nxt, 1 - pe):
                c.start()
        for c in w2_copies(cur, pe):
            c.wait()

    o = lax.dot_general(a_ref[...], wbuf[pe], (((1,), (1,)), ((), ())),
                        preferred_element_type=_f32,
                        precision=lax.Precision.DEFAULT)
    out_ref[...] = pltpu.pack_elementwise([o[:, :HW], o[:, HW:]],
                                          packed_dtype=jnp.bfloat16)


_gemm2 = pl.pallas_call(
    _gemm2_body,
    grid_spec=pltpu.PrefetchScalarGridSpec(
        num_scalar_prefetch=1,
        grid=(NBLK,),
        in_specs=[
            pl.BlockSpec((BT, INTER), lambda i, m: (i, 0)),
            pl.BlockSpec(memory_space=pl.ANY),
        ],
        out_specs=pl.BlockSpec((BT, HW), lambda i, m: (i, 0)),
        scratch_shapes=[pltpu.VMEM((2, HID, INTER), _f32),
                        pltpu.SemaphoreType.DMA((2, 2))],
    ),
    out_shape=jax.ShapeDtypeStruct((S, HW), jnp.int32),
)


BC = 256


def _combine_body(wk_ref, g0_ref, g1_ref, out_ref):
    wk = wk_ref[...]
    g0 = g0_ref[...]
    g1 = g1_ref[...]
    w0 = wk[:, 0:1]
    w1 = wk[:, 1:2]

    def u(x, idx):
        return pltpu.unpack_elementwise(x, index=idx,
                                        packed_dtype=jnp.bfloat16,
                                        unpacked_dtype=_f32)

    lo = u(g0, 0) * w0 + u(g1, 0) * w1
    hi = u(g0, 1) * w0 + u(g1, 1) * w1
    out_ref[...] = jnp.concatenate([lo, hi], axis=1)


_combine = pl.pallas_call(
    _combine_body,
    compiler_params=pltpu.CompilerParams(dimension_semantics=("parallel",)),
    grid=(T // BC,),
    in_specs=[pl.BlockSpec((BC, K), lambda i: (i, 0)),
              pl.BlockSpec((BC, HW), lambda i: (i, 0)),
              pl.BlockSpec((BC, HW), lambda i: (i + T // BC, 0))],
    out_specs=pl.BlockSpec((BC, HID), lambda i: (i, 0)),
    out_shape=jax.ShapeDtypeStruct((T, HID), _f32),
)


@functools.cache
def _sc_kernels():
    mesh = plsc.VectorSubcoreMesh(core_axis_name="c", subcore_axis_name="s",
                                  num_cores=2, num_subcores=16)

    @functools.partial(
        pl.kernel, mesh=mesh,
        out_type=jax.ShapeDtypeStruct((S, HW), jnp.int32),
        scratch_types=[pltpu.VMEM((NCH, CH), jnp.int32),
                       pltpu.VMEM((CH, HW), jnp.int32)],
    )
    def dispatch(x_hbm, d_hbm, xg_hbm, idx_v, rows_v):
        wid = lax.axis_index("s") * 2 + lax.axis_index("c")
        pltpu.sync_copy(d_hbm.at[wid], idx_v)
        base = wid * PW

        @pl.loop(0, NCH)
        def _(j):
            tok = lax.rem(base + j * CH, T)
            pltpu.sync_copy(x_hbm.at[pl.ds(tok, CH)], rows_v)
            pltpu.sync_copy(rows_v, xg_hbm.at[idx_v.at[j]])

    @functools.partial(
        pl.kernel, mesh=mesh,
        out_type=jax.ShapeDtypeStruct((P, HW), jnp.int32),
        scratch_types=[pltpu.VMEM((NCH, CH), jnp.int32),
                       pltpu.VMEM((CH, HW), jnp.int32)],
    )
    def gather_rows(r_hbm, d_hbm, g_hbm, idx_v, rows_v):
        wid = lax.axis_index("s") * 2 + lax.axis_index("c")
        pltpu.sync_copy(d_hbm.at[wid], idx_v)
        base = wid * PW

        @pl.loop(0, NCH)
        def _(j):
            pltpu.sync_copy(r_hbm.at[idx_v.at[j]], rows_v)
            pltpu.sync_copy(rows_v, g_hbm.at[pl.ds(base + j * CH, CH)])

    return dispatch, gather_rows


def kernel(hidden_states, router_logits, w13, w2):
    _dispatch, _gather_rows = _sc_kernels()
    dest, wk, be, xp = _routing(router_logits, hidden_states)
    dest3 = dest.reshape(NW, NCH, CH)
    xg = _dispatch(xp, dest3)
    h = _gemm1(be, xg, w13)
    rows = _gemm2(be, h, w2)
    g = _gather_rows(rows, dest3)
    return _combine(wk, g, g)


# speedup vs baseline: 1.0149x; 1.0149x over previous
import functools

import jax
import jax.numpy as jnp
from jax import lax
from jax.experimental import pallas as pl
from jax.experimental.pallas import tpu as pltpu
from jax.experimental.pallas import tpu_sc as plsc

T = 2048
E = 8
HID = 2048
INTER = 1408
K = 2
BT = 256
P = T * K
S = P + E * BT
NBLK = S // BT
NBPAD = 32
NW = 32
PW = P // NW
CH = 16
NCH = PW // CH
HW = HID // 2

_f32 = jnp.float32


def _routing_body(logits_ref, x_ref, dest_ref, wk_ref, be_ref, xp_ref):
    x = x_ref[...]
    xp_ref[...] = pltpu.pack_elementwise([x[:, :HW], x[:, HW:]],
                                         packed_dtype=jnp.bfloat16)
    l = logits_ref[...]
    io = lax.broadcasted_iota(jnp.int32, (T, E), 1)
    m1 = jnp.max(l, axis=1, keepdims=True)
    sel1 = io == jnp.min(jnp.where(l == m1, io, E), axis=1, keepdims=True)
    lm = jnp.where(sel1, -jnp.inf, l)
    m2 = jnp.max(lm, axis=1, keepdims=True)
    sel2 = io == jnp.min(jnp.where(lm == m2, io, E), axis=1, keepdims=True)
    w0 = jax.nn.sigmoid(m1 - m2)
    wk_ref[...] = jnp.concatenate([w0, 1.0 - w0], axis=1)

    oh = jnp.concatenate([sel1.astype(_f32), sel2.astype(_f32)], axis=0)
    c = 128
    ri = lax.broadcasted_iota(jnp.int32, (c, c), 0)
    ci = lax.broadcasted_iota(jnp.int32, (c, c), 1)
    ltri = (ri > ci).astype(_f32)
    ones_row = jnp.ones((1, c), _f32)
    running = jnp.zeros((1, E), _f32)
    chunks = []
    for i in range(P // c):
        oc = lax.slice(oh, (i * c, 0), ((i + 1) * c, E))
        chunks.append(jnp.dot(ltri, oc, preferred_element_type=_f32) + running)
        running = running + jnp.dot(ones_row, oc, preferred_element_type=_f32)
    prefix = jnp.concatenate(chunks, axis=0)
    cap = jnp.ceil(running / BT) * BT
    a_lt_b = (lax.broadcasted_iota(jnp.int32, (E, E), 0)
              < lax.broadcasted_iota(jnp.int32, (E, E), 1)).astype(_f32)
    start = jnp.dot(cap, a_lt_b, preferred_element_type=_f32)
    dest = jnp.sum(oh * (start + prefix), axis=1, keepdims=True)
    dest_ref[...] = dest.astype(jnp.int32)
    posb = (lax.broadcasted_iota(jnp.int32, (NBPAD, E), 0) * BT).astype(_f32)
    eiof = lax.broadcasted_iota(jnp.int32, (NBPAD, E), 1).astype(_f32)
    bef = jnp.sum((start <= posb).astype(_f32), axis=1, keepdims=True) - 1.0
    present = (running > 0).astype(_f32)
    rank = jnp.dot(present, a_lt_b, preferred_element_type=_f32)
    runidx = jnp.sum(present * (eiof < bef).astype(_f32), axis=1, keepdims=True)
    parity = runidx - 2.0 * jnp.floor(runidx / 2.0)
    startb = jnp.sum((eiof == bef).astype(_f32) * start, axis=1, keepdims=True)
    posb0 = posb[:, 0:1]
    tot = jnp.sum(cap, axis=1, keepdims=True)
    used = (posb0 < tot).astype(_f32)
    first = ((startb == posb0).astype(_f32)) * used
    sel_nxt = ((rank == runidx + 1.0).astype(_f32)) * present
    has = jnp.sum(sel_nxt, axis=1, keepdims=True)
    nxt = jnp.sum(sel_nxt * eiof, axis=1, keepdims=True) - (1.0 - has)
    zero = jnp.zeros((NBPAD, 3), _f32)
    meta = jnp.concatenate([bef, first, parity, nxt, used, zero], axis=1)
    be_ref[...] = meta.astype(jnp.int32)


_routing = pl.pallas_call(
    _routing_body,
    out_shape=(jax.ShapeDtypeStruct((P, 1), jnp.int32),
               jax.ShapeDtypeStruct((T, K), _f32),
               jax.ShapeDtypeStruct((NBPAD, 8), jnp.int32),
               jax.ShapeDtypeStruct((T, HW), jnp.int32)),
)


def _gemm1_body(m_ref, xg_ref, w13_hbm, a_ref, wbuf, sems):
    i = pl.program_id(0)
    cur = m_ref[i, 0]
    first = m_ref[i, 1]
    pe = m_ref[i, 2]
    nxt = m_ref[i, 3]

    def w13_copies(e, slot):
        return [pltpu.make_async_copy(w13_hbm.at[e, pl.ds(lo, INTER)],
                                      wbuf.at[slot, pl.ds(lo, INTER)],
                                      sems.at[slot, q])
                for q, lo in ((0, 0), (1, INTER))]

    @pl.when(i == 0)
    def _():
        for c in w13_copies(cur, 0):
            c.start()

    @pl.when(first == 1)
    def _():
        @pl.when(nxt >= 0)
        def _():
            for c in w13_copies(nxt, 1 - pe):
                c.start()
        for c in w13_copies(cur, pe):
            c.wait()

    xi = xg_ref[...]
    x = jnp.concatenate(
        [pltpu.unpack_elementwise(xi, index=0, packed_dtype=jnp.bfloat16,
                                  unpacked_dtype=_f32),
         pltpu.unpack_elementwise(xi, index=1, packed_dtype=jnp.bfloat16,
                                  unpacked_dtype=_f32)], axis=1)
    h = lax.dot_general(x, wbuf[pe], (((1,), (1,)), ((), ())),
                        preferred_element_type=_f32,
                        precision=lax.Precision.DEFAULT)
    g = h[:, :INTER]
    u = h[:, INTER:]
    a_ref[...] = (g * jax.nn.sigmoid(g) * u).astype(jnp.bfloat16)


_gemm1 = pl.pallas_call(
    _gemm1_body,
    grid_spec=pltpu.PrefetchScalarGridSpec(
        num_scalar_prefetch=1,
        grid=(NBLK,),
        in_specs=[
            pl.BlockSpec((BT, HW), lambda i, m: (i, 0)),
            pl.BlockSpec(memory_space=pl.ANY),
        ],
        out_specs=pl.BlockSpec((BT, INTER), lambda i, m: (i, 0)),
        scratch_shapes=[pltpu.VMEM((2, 2 * INTER, HID), _f32),
                        pltpu.SemaphoreType.DMA((2, 2))],
    ),
    out_shape=jax.ShapeDtypeStruct((S, INTER), jnp.bfloat16),
)


def _gemm2_body(m_ref, a_ref, w2_hbm, out_ref, wbuf, sems):
    i = pl.program_id(0)
    cur = m_ref[i, 0]
    first = m_ref[i, 1]
    pe = m_ref[i, 2]
    nxt = m_ref[i, 3]

    def w2_copies(e, slot):
        return [pltpu.make_async_copy(w2_hbm.at[e, pl.ds(lo, HID // 2)],
                                      wbuf.at[slot, pl.ds(lo, HID // 2)],
                                      sems.at[slot, q])
                for q, lo in ((0, 0), (1, HID // 2))]

    @pl.when(i == 0)
    def _():
        for c in w2_copies(cur, 0):
            c.start()

    @pl.when(first == 1)
    def _():
        @pl.when(nxt >= 0)
        def _():
            for c in w2_copies(nxt, 1 - pe):
                c.start()
        for c in w2_copies(cur, pe):
            c.wait()

    o = lax.dot_general(a_ref[...], wbuf[pe], (((1,), (1,)), ((), ())),
                        preferred_element_type=_f32,
                        precision=lax.Precision.DEFAULT)
    out_ref[...] = pltpu.pack_elementwise([o[:, :HW], o[:, HW:]],
                                          packed_dtype=jnp.bfloat16)


_gemm2 = pl.pallas_call(
    _gemm2_body,
    grid_spec=pltpu.PrefetchScalarGridSpec(
        num_scalar_prefetch=1,
        grid=(NBLK,),
        in_specs=[
            pl.BlockSpec((BT, INTER), lambda i, m: (i, 0)),
            pl.BlockSpec(memory_space=pl.ANY),
        ],
        out_specs=pl.BlockSpec((BT, HW), lambda i, m: (i, 0)),
        scratch_shapes=[pltpu.VMEM((2, HID, INTER), _f32),
                        pltpu.SemaphoreType.DMA((2, 2))],
    ),
    out_shape=jax.ShapeDtypeStruct((S, HW), jnp.int32),
)


BC = 256


def _combine_body(wk_ref, g0_ref, g1_ref, out_ref):
    wk = wk_ref[...]
    g0 = g0_ref[...]
    g1 = g1_ref[...]
    w0 = wk[:, 0:1]
    w1 = wk[:, 1:2]

    def u(x, idx):
        return pltpu.unpack_elementwise(x, index=idx,
                                        packed_dtype=jnp.bfloat16,
                                        unpacked_dtype=_f32)

    lo = u(g0, 0) * w0 + u(g1, 0) * w1
    hi = u(g0, 1) * w0 + u(g1, 1) * w1
    out_ref[...] = jnp.concatenate([lo, hi], axis=1)


_combine = pl.pallas_call(
    _combine_body,
    compiler_params=pltpu.CompilerParams(dimension_semantics=("parallel",)),
    grid=(T // BC,),
    in_specs=[pl.BlockSpec((BC, K), lambda i: (i, 0)),
              pl.BlockSpec((BC, HW), lambda i: (i, 0)),
              pl.BlockSpec((BC, HW), lambda i: (i + T // BC, 0))],
    out_specs=pl.BlockSpec((BC, HID), lambda i: (i, 0)),
    out_shape=jax.ShapeDtypeStruct((T, HID), _f32),
)


@functools.cache
def _sc_kernels():
    mesh = plsc.VectorSubcoreMesh(core_axis_name="c", subcore_axis_name="s",
                                  num_cores=2, num_subcores=16)

    @functools.partial(
        pl.kernel, mesh=mesh,
        out_type=jax.ShapeDtypeStruct((S, HW), jnp.int32),
        scratch_types=[pltpu.VMEM((NCH, CH), jnp.int32),
                       pltpu.VMEM((CH, HW), jnp.int32),
                       pltpu.VMEM((CH, HW), jnp.int32),
                       pltpu.SemaphoreType.DMA((2,)),
                       pltpu.SemaphoreType.DMA((2,))],
    )
    def dispatch(x_hbm, d_hbm, xg_hbm, idx_v, r0, r1, si, so):
        wid = lax.axis_index("s") * 2 + lax.axis_index("c")
        pltpu.sync_copy(d_hbm.at[wid], idx_v)
        base = wid * PW
        rv = (r0, r1)

        def cin(j, b):
            tok = lax.rem(base + j * CH, T)
            return pltpu.make_async_copy(x_hbm.at[pl.ds(tok, CH)], rv[b],
                                         si.at[b])

        def cout(j, b):
            return pltpu.make_async_copy(rv[b], xg_hbm.at[idx_v.at[j]],
                                         so.at[b])

        cin(0, 0).start()
        for j in range(NCH):
            b = j % 2
            cin(j, b).wait()
            if j + 1 < NCH:
                if j >= 1:
                    cout(j - 1, 1 - b).wait()
                cin(j + 1, 1 - b).start()
            cout(j, b).start()
        cout(NCH - 2, NCH % 2).wait()
        cout(NCH - 1, 1 - NCH % 2).wait()

    @functools.partial(
        pl.kernel, mesh=mesh,
        out_type=jax.ShapeDtypeStruct((P, HW), jnp.int32),
        scratch_types=[pltpu.VMEM((NCH, CH), jnp.int32),
                       pltpu.VMEM((CH, HW), jnp.int32),
                       pltpu.VMEM((CH, HW), jnp.int32),
                       pltpu.SemaphoreType.DMA((2,)),
                       pltpu.SemaphoreType.DMA((2,))],
    )
    def gather_rows(r_hbm, d_hbm, g_hbm, idx_v, r0, r1, si, so):
        wid = lax.axis_index("s") * 2 + lax.axis_index("c")
        pltpu.sync_copy(d_hbm.at[wid], idx_v)
        base = wid * PW
        rv = (r0, r1)

        def cin(j, b):
            return pltpu.make_async_copy(r_hbm.at[idx_v.at[j]], rv[b],
                                         si.at[b])

        def cout(j, b):
            return pltpu.make_async_copy(rv[b],
                                         g_hbm.at[pl.ds(base + j * CH, CH)],
                                         so.at[b])

        cin(0, 0).start()
        for j in range(NCH):
            b = j % 2
            cin(j, b).wait()
            if j + 1 < NCH:
                if j >= 1:
                    cout(j - 1, 1 - b).wait()
                cin(j + 1, 1 - b).start()
            cout(j, b).start()
        cout(NCH - 2, NCH % 2).wait()
        cout(NCH - 1, 1 - NCH % 2).wait()

    return dispatch, gather_rows


def kernel(hidden_states, router_logits, w13, w2):
    _dispatch, _gather_rows = _sc_kernels()
    dest, wk, be, xp = _routing(router_logits, hidden_states)
    dest3 = dest.reshape(NW, NCH, CH)
    xg = _dispatch(xp, dest3)
    h = _gemm1(be, xg, w13)
    rows = _gemm2(be, h, w2)
    g = _gather_rows(rows, dest3)
    return _combine(wk, g, g)
